# Initial kernel scaffold; baseline (speedup 1.0000x reference)
import jax
import jax.numpy as jnp
from jax.experimental import pallas as pl


def kernel(x, edge_index, batch, W_in, b_in, W_c0, b_c0, gamma0, beta0, W_c1, b_c1, gamma1, beta1, W_c2, b_c2, gamma2, beta2, fc1_W, fc1_b, fc2_W, fc2_b):
    raise NotImplementedError("write your pallas kernel here")


# speedup vs baseline: 10.0646x; 10.0646x over previous
import jax
import jax.numpy as jnp
from jax import lax
from jax.experimental import pallas as pl
from jax.experimental.pallas import tpu as pltpu
from jax.experimental.pallas import tpu_sc as plsc

N = 10000
E = 320000
DH = 128
NG = 64
DOUT = 16

NC = 2
NS = 16
NW = NC * NS
EPT = E // NW
CB = 80
NCHUNK = EPT // CB
NWT = 10
WRT = N // NWT
DW = 128

BM = 1000
NBM = N // BM

_mesh = plsc.VectorSubcoreMesh(core_axis_name="c", subcore_axis_name="s",
                               num_cores=NC, num_subcores=NS)


def _deg_body(coli, zeros16, ones16, out, cidx, ones_v, deg, sem):
    c = lax.axis_index("c")
    s = lax.axis_index("s")
    wid = c * NS + s

    @pl.when(s < NWT)
    def _():
        pltpu.sync_copy(zeros16.at[pl.ds(s * WRT, WRT)],
                        deg.at[pl.ds(s * WRT, WRT)])

    pltpu.sync_copy(ones16, ones_v)
    plsc.subcore_barrier()

    def chunk(k, carry):
        base = wid * EPT + k * CB
        pltpu.sync_copy(coli.at[pl.ds(base, CB)], cidx)
        pltpu.sync_copy(ones_v, deg.at[cidx], add=True)
        return carry

    lax.fori_loop(0, NCHUNK, chunk, 0)
    plsc.subcore_barrier()

    @pl.when(s < NWT)
    def _():
        pltpu.sync_copy(deg.at[pl.ds(s * WRT, WRT)],
                        out.at[c, pl.ds(s * WRT, WRT)])


_deg_call = pl.kernel(
    _deg_body,
    out_type=jax.ShapeDtypeStruct((NC, N, DW), jnp.float32),
    mesh=_mesh,
    scratch_types=[
        pltpu.VMEM((CB,), jnp.int32),
        pltpu.VMEM((CB, DW), jnp.float32),
        pltpu.VMEM_SHARED((N, DW), jnp.float32),
        pltpu.SemaphoreType.DMA,
    ],
)


def _seg_body(hws, rowi, coli, zeros, out, ridx, cidx, rows, agg, sem):
    c = lax.axis_index("c")
    s = lax.axis_index("s")
    wid = c * NS + s

    @pl.when(s < NWT)
    def _():
        pltpu.sync_copy(zeros.at[pl.ds(s * WRT, WRT)],
                        agg.at[pl.ds(s * WRT, WRT)])

    plsc.subcore_barrier()

    def chunk(k, carry):
        base = wid * EPT + k * CB
        pltpu.sync_copy(rowi.at[pl.ds(base, CB)], ridx)
        pltpu.sync_copy(coli.at[pl.ds(base, CB)], cidx)
        pltpu.async_copy(hws.at[ridx], rows, sem).wait()
        pltpu.sync_copy(rows, agg.at[cidx], add=True)
        return carry

    lax.fori_loop(0, NCHUNK, chunk, 0)
    plsc.subcore_barrier()

    @pl.when(s < NWT)
    def _():
        pltpu.sync_copy(agg.at[pl.ds(s * WRT, WRT)],
                        out.at[c, pl.ds(s * WRT, WRT)])


_seg_call = pl.kernel(
    _seg_body,
    out_type=jax.ShapeDtypeStruct((NC, N, DH), jnp.float32),
    mesh=_mesh,
    scratch_types=[
        pltpu.VMEM((CB,), jnp.int32),
        pltpu.VMEM((CB,), jnp.int32),
        pltpu.VMEM((CB, DH), jnp.float32),
        pltpu.VMEM_SHARED((N, DH), jnp.float32),
        pltpu.SemaphoreType.DMA,
    ],
)


def _mm_body(a_ref, w_ref, b_ref, d_ref, o_ref):
    acc = lax.dot_general(
        a_ref[...], w_ref[...], (((1,), (0,)), ((), ())),
        preferred_element_type=jnp.float32,
        precision=lax.Precision.HIGHEST,
    )
    o_ref[...] = (acc + b_ref[...]) * d_ref[...]


def _mm(a, w, bias, dscale):
    n, k = a.shape
    m = w.shape[1]
    return pl.pallas_call(
        _mm_body,
        grid=(n // BM,),
        in_specs=[
            pl.BlockSpec((BM, k), lambda i: (i, 0)),
            pl.BlockSpec((k, m), lambda i: (0, 0)),
            pl.BlockSpec((1, m), lambda i: (0, 0)),
            pl.BlockSpec((BM, 1), lambda i: (i, 0)),
        ],
        out_specs=pl.BlockSpec((BM, m), lambda i: (i, 0)),
        out_shape=jax.ShapeDtypeStruct((n, m), jnp.float32),
    )(a, w, bias, dscale)


def _dinv_body(degp_ref, o_ref):
    deg = degp_ref[0] + degp_ref[1] + 2.0
    o_ref[...] = lax.rsqrt(deg)


def _dinv_call(degp):
    return pl.pallas_call(
        _dinv_body,
        out_shape=jax.ShapeDtypeStruct((N, DW), jnp.float32),
    )(degp)


def _post_body(agg_ref, hws_ref, d_ref, b_ref, g_ref, be_ref, h_ref, o_ref,
               t_scr, s_scr, ss_scr):
    p = pl.program_id(0)
    i = pl.program_id(1)

    @pl.when(p == 0)
    def _():
        t = d_ref[...] * (agg_ref[0] + agg_ref[1] + 2.0 * hws_ref[...]) + b_ref[...]
        t_scr[pl.ds(i * BM, BM), :] = t
        ps = jnp.sum(t, axis=0, keepdims=True)
        pss = jnp.sum(t * t, axis=0, keepdims=True)

        @pl.when(i == 0)
        def _():
            s_scr[0:1, :] = ps
            ss_scr[0:1, :] = pss

        @pl.when(i > 0)
        def _():
            s_scr[0:1, :] += ps
            ss_scr[0:1, :] += pss

    @pl.when(p == 1)
    def _():
        m = s_scr[0:1, :] / N
        v = ss_scr[0:1, :] / N - m * m
        t = t_scr[pl.ds(i * BM, BM), :]
        bn = (t - m) * lax.rsqrt(v + 1e-5) * g_ref[...] + be_ref[...]
        o_ref[...] = jnp.maximum(bn, 0.0) + h_ref[...]


def _post(agg, hws, dinv, b, g, be, h):
    return pl.pallas_call(
        _post_body,
        grid=(2, NBM),
        in_specs=[
            pl.BlockSpec((NC, BM, DH), lambda p, i: (0, i, 0)),
            pl.BlockSpec((BM, DH), lambda p, i: (i, 0)),
            pl.BlockSpec((BM, 1), lambda p, i: (i, 0)),
            pl.BlockSpec((1, DH), lambda p, i: (0, 0)),
            pl.BlockSpec((1, DH), lambda p, i: (0, 0)),
            pl.BlockSpec((1, DH), lambda p, i: (0, 0)),
            pl.BlockSpec((BM, DH), lambda p, i: (i, 0)),
        ],
        out_specs=pl.BlockSpec((BM, DH), lambda p, i: (i, 0)),
        out_shape=jax.ShapeDtypeStruct((N, DH), jnp.float32),
        scratch_shapes=[
            pltpu.VMEM((N, DH), jnp.float32),
            pltpu.VMEM((8, DH), jnp.float32),
            pltpu.VMEM((8, DH), jnp.float32),
        ],
    )(agg, hws, dinv, b, g, be, h)


def _tail_body(b3_ref, h_ref, f1w_ref, f1b_ref, f2w_ref, f2b_ref, o_ref,
               ps_scr, ct_scr):
    i = pl.program_id(0)
    bb = b3_ref[0]
    gid = lax.broadcasted_iota(jnp.int32, (NG, BM), 0)
    P = (jnp.broadcast_to(bb, (NG, BM)) == gid).astype(jnp.float32)
    part = lax.dot_general(
        P, h_ref[...], (((1,), (0,)), ((), ())),
        preferred_element_type=jnp.float32,
        precision=lax.Precision.HIGHEST,
    )
    cnt = jnp.sum(P, axis=1, keepdims=True)

    @pl.when(i == 0)
    def _():
        ps_scr[...] = part
        ct_scr[...] = jnp.broadcast_to(cnt, (NG, DH))

    @pl.when(i > 0)
    def _():
        ps_scr[...] += part
        ct_scr[...] += jnp.broadcast_to(cnt, (NG, DH))

    @pl.when(i == NBM - 1)
    def _():
        pooled = ps_scr[...] / jnp.maximum(ct_scr[...], 1.0)
        o1 = lax.dot_general(
            pooled, f1w_ref[...], (((1,), (0,)), ((), ())),
            preferred_element_type=jnp.float32,
            precision=lax.Precision.HIGHEST,
        )
        o1 = jnp.maximum(o1 + f1b_ref[...], 0.0)
        o2 = lax.dot_general(
            o1, f2w_ref[...], (((1,), (0,)), ((), ())),
            preferred_element_type=jnp.float32,
            precision=lax.Precision.HIGHEST,
        )
        o_ref[...] = o2 + f2b_ref[...]


def _tail(batch3, h, f1w, f1b, f2w, f2b):
    return pl.pallas_call(
        _tail_body,
        grid=(NBM,),
        in_specs=[
            pl.BlockSpec((1, 1, BM), lambda i: (i, 0, 0)),
            pl.BlockSpec((BM, DH), lambda i: (i, 0)),
            pl.BlockSpec((DH, DH), lambda i: (0, 0)),
            pl.BlockSpec((1, DH), lambda i: (0, 0)),
            pl.BlockSpec((DH, DOUT), lambda i: (0, 0)),
            pl.BlockSpec((1, DOUT), lambda i: (0, 0)),
        ],
        out_specs=pl.BlockSpec((NG, DOUT), lambda i: (0, 0)),
        out_shape=jax.ShapeDtypeStruct((NG, DOUT), jnp.float32),
        scratch_shapes=[
            pltpu.VMEM((NG, DH), jnp.float32),
            pltpu.VMEM((NG, DH), jnp.float32),
        ],
    )(batch3, h, f1w, f1b, f2w, f2b)


def kernel(x, edge_index, batch, W_in, b_in,
           W_c0, b_c0, gamma0, beta0,
           W_c1, b_c1, gamma1, beta1,
           W_c2, b_c2, gamma2, beta2,
           fc1_W, fc1_b, fc2_W, fc2_b):
    row = edge_index[0]
    col = edge_index[1]
    zeros_nd = jnp.zeros((N, DH), jnp.float32)
    zeros_nw = jnp.zeros((N, DW), jnp.float32)
    ones_cb = jnp.ones((CB, DW), jnp.float32)
    ones_n1 = jnp.ones((N, 1), jnp.float32)
    zero_b = jnp.zeros((1, DH), jnp.float32)

    degp = _deg_call(col, zeros_nw, ones_cb)
    dinv = _dinv_call(degp)[:, 0:1]

    h = _mm(x, W_in, b_in.reshape(1, DH), ones_n1)

    layers = [(W_c0, b_c0, gamma0, beta0),
              (W_c1, b_c1, gamma1, beta1),
              (W_c2, b_c2, gamma2, beta2)]
    for (W, b, g, be) in layers:
        hws = _mm(h, W, zero_b, dinv)
        agg = _seg_call(hws, row, col, zeros_nd)
        h = _post(agg, hws, dinv, b.reshape(1, DH), g.reshape(1, DH),
                  be.reshape(1, DH), h)

    batch3 = batch.reshape(NBM, 1, BM)
    out = _tail(batch3, h, fc1_W, fc1_b.reshape(1, DH),
                fc2_W, fc2_b.reshape(1, DOUT))
    return out


# speedup vs baseline: 21.0704x; 2.0935x over previous
import jax
import jax.numpy as jnp
from jax import lax
from jax.experimental import pallas as pl
from jax.experimental.pallas import tpu as pltpu
from jax.experimental.pallas import tpu_sc as plsc

N = 10000
E = 320000
DH = 128
NG = 64
DOUT = 16

NC = 2
NS = 16
NW = NC * NS
EPT = E // NW
CB = 40
NCHUNK = EPT // CB
G = 5
NGRP = NCHUNK // G
NWT = 10
WRT = N // NWT
DW = 128

BM = 1000
NBM = N // BM

_mesh = plsc.VectorSubcoreMesh(core_axis_name="c", subcore_axis_name="s",
                               num_cores=NC, num_subcores=NS)


def _deg_body(colr, zeros16, ones16, out, cidx, ones_v, deg, sem):
    c = lax.axis_index("c")
    s = lax.axis_index("s")
    wid = c * NS + s
    pltpu.sync_copy(colr.at[pl.ds(wid * EPT, EPT)], cidx)

    @pl.when(s < NWT)
    def _():
        pltpu.sync_copy(zeros16.at[pl.ds(s * WRT, WRT)],
                        deg.at[pl.ds(s * WRT, WRT)])

    pltpu.sync_copy(ones16, ones_v)
    plsc.subcore_barrier()

    def grp(g, carry):
        for j in range(G):
            pltpu.async_copy(
                ones_v, deg.at[cidx.at[pl.ds((g * G + j) * CB, CB)]],
                sem, add=True)
        for j in range(G):
            pltpu.make_async_copy(
                ones_v, deg.at[cidx.at[pl.ds(0, CB)]], sem).wait()
        return carry

    lax.fori_loop(0, NGRP, grp, 0)
    plsc.subcore_barrier()

    @pl.when(s < NWT)
    def _():
        pltpu.sync_copy(deg.at[pl.ds(s * WRT, WRT)],
                        out.at[c, pl.ds(s * WRT, WRT)])


_deg_call = pl.kernel(
    _deg_body,
    out_type=jax.ShapeDtypeStruct((NC, N, DW), jnp.float32),
    mesh=_mesh,
    scratch_types=[
        pltpu.VMEM((EPT,), jnp.int32),
        pltpu.VMEM((CB, DW), jnp.float32),
        pltpu.VMEM_SHARED((N, DW), jnp.float32),
        pltpu.SemaphoreType.DMA,
    ],
)


def _seg_body(hws, rowr, colr, zeros, out, ridx, cidx,
              r0, r1, r2, r3, r4, agg,
              g0, g1, g2, g3, g4, s0, s1, s2, s3, s4):
    rows = [r0, r1, r2, r3, r4]
    gsem = [g0, g1, g2, g3, g4]
    ssem = [s0, s1, s2, s3, s4]
    c = lax.axis_index("c")
    s = lax.axis_index("s")
    wid = c * NS + s
    pltpu.sync_copy(rowr.at[pl.ds(wid * EPT, EPT)], ridx)
    pltpu.sync_copy(colr.at[pl.ds(wid * EPT, EPT)], cidx)

    @pl.when(s < NWT)
    def _():
        pltpu.sync_copy(zeros.at[pl.ds(s * WRT, WRT)],
                        agg.at[pl.ds(s * WRT, WRT)])

    plsc.subcore_barrier()

    for j in range(G):
        pltpu.async_copy(hws.at[ridx.at[pl.ds(j * CB, CB)]], rows[j], gsem[j])

    def grp(g, carry):
        for j in range(G):
            k = g * G + j
            pltpu.make_async_copy(hws.at[ridx.at[pl.ds(k * CB, CB)]],
                                  rows[j], gsem[j]).wait()
            pltpu.async_copy(rows[j], agg.at[cidx.at[pl.ds(k * CB, CB)]],
                             ssem[j], add=True)

        @pl.when(g < NGRP - 1)
        def _():
            for j in range(G):
                k = g * G + j
                pltpu.make_async_copy(rows[j],
                                      agg.at[cidx.at[pl.ds(k * CB, CB)]],
                                      ssem[j]).wait()
                pltpu.async_copy(hws.at[ridx.at[pl.ds((k + G) * CB, CB)]],
                                 rows[j], gsem[j])

        return carry

    lax.fori_loop(0, NGRP, grp, 0)
    for j in range(G):
        pltpu.make_async_copy(rows[j], agg.at[cidx.at[pl.ds(0, CB)]],
                              ssem[j]).wait()
    plsc.subcore_barrier()

    @pl.when(s < NWT)
    def _():
        pltpu.sync_copy(agg.at[pl.ds(s * WRT, WRT)],
                        out.at[c, pl.ds(s * WRT, WRT)])


_seg_call = pl.kernel(
    _seg_body,
    out_type=jax.ShapeDtypeStruct((NC, N, DH), jnp.float32),
    mesh=_mesh,
    scratch_types=[
        pltpu.VMEM((EPT,), jnp.int32),
        pltpu.VMEM((EPT,), jnp.int32),
    ] + [pltpu.VMEM((CB, DH), jnp.float32) for _ in range(G)] + [
        pltpu.VMEM_SHARED((N, DH), jnp.float32),
    ] + [pltpu.SemaphoreType.DMA for _ in range(2 * G)],
)


def _mm_body(a_ref, w_ref, b_ref, d_ref, o_ref):
    acc = lax.dot_general(
        a_ref[...], w_ref[...], (((1,), (0,)), ((), ())),
        preferred_element_type=jnp.float32,
        precision=lax.Precision.HIGHEST,
    )
    o_ref[...] = (acc + b_ref[...]) * d_ref[...]


def _mm(a, w, bias, dscale):
    n, k = a.shape
    m = w.shape[1]
    return pl.pallas_call(
        _mm_body,
        grid=(n // BM,),
        in_specs=[
            pl.BlockSpec((BM, k), lambda i: (i, 0)),
            pl.BlockSpec((k, m), lambda i: (0, 0)),
            pl.BlockSpec((1, m), lambda i: (0, 0)),
            pl.BlockSpec((BM, 1), lambda i: (i, 0)),
        ],
        out_specs=pl.BlockSpec((BM, m), lambda i: (i, 0)),
        out_shape=jax.ShapeDtypeStruct((n, m), jnp.float32),
    )(a, w, bias, dscale)


def _dinv_body(degp_ref, o_ref):
    deg = degp_ref[0] + degp_ref[1] + 2.0
    o_ref[...] = lax.rsqrt(deg)


def _dinv_call(degp):
    return pl.pallas_call(
        _dinv_body,
        out_shape=jax.ShapeDtypeStruct((N, DW), jnp.float32),
    )(degp)


def _post_body(agg_ref, hws_ref, d_ref, b_ref, g_ref, be_ref, h_ref, o_ref,
               t_scr, s_scr, ss_scr):
    p = pl.program_id(0)
    i = pl.program_id(1)

    @pl.when(p == 0)
    def _():
        t = d_ref[...] * (agg_ref[0] + agg_ref[1] + 2.0 * hws_ref[...]) + b_ref[...]
        t_scr[pl.ds(i * BM, BM), :] = t
        ps = jnp.sum(t, axis=0, keepdims=True)
        pss = jnp.sum(t * t, axis=0, keepdims=True)

        @pl.when(i == 0)
        def _():
            s_scr[0:1, :] = ps
            ss_scr[0:1, :] = pss

        @pl.when(i > 0)
        def _():
            s_scr[0:1, :] += ps
            ss_scr[0:1, :] += pss

    @pl.when(p == 1)
    def _():
        m = s_scr[0:1, :] / N
        v = ss_scr[0:1, :] / N - m * m
        t = t_scr[pl.ds(i * BM, BM), :]
        bn = (t - m) * lax.rsqrt(v + 1e-5) * g_ref[...] + be_ref[...]
        o_ref[...] = jnp.maximum(bn, 0.0) + h_ref[...]


def _post(agg, hws, dinv, b, g, be, h):
    return pl.pallas_call(
        _post_body,
        grid=(2, NBM),
        in_specs=[
            pl.BlockSpec((NC, BM, DH), lambda p, i: (0, i, 0)),
            pl.BlockSpec((BM, DH), lambda p, i: (i, 0)),
            pl.BlockSpec((BM, 1), lambda p, i: (i, 0)),
            pl.BlockSpec((1, DH), lambda p, i: (0, 0)),
            pl.BlockSpec((1, DH), lambda p, i: (0, 0)),
            pl.BlockSpec((1, DH), lambda p, i: (0, 0)),
            pl.BlockSpec((BM, DH), lambda p, i: (i, 0)),
        ],
        out_specs=pl.BlockSpec((BM, DH), lambda p, i: (i, 0)),
        out_shape=jax.ShapeDtypeStruct((N, DH), jnp.float32),
        scratch_shapes=[
            pltpu.VMEM((N, DH), jnp.float32),
            pltpu.VMEM((8, DH), jnp.float32),
            pltpu.VMEM((8, DH), jnp.float32),
        ],
    )(agg, hws, dinv, b, g, be, h)


def _tail_body(b3_ref, h_ref, f1w_ref, f1b_ref, f2w_ref, f2b_ref, o_ref,
               ps_scr, ct_scr):
    i = pl.program_id(0)
    bb = b3_ref[0]
    gid = lax.broadcasted_iota(jnp.int32, (NG, BM), 0)
    P = (jnp.broadcast_to(bb, (NG, BM)) == gid).astype(jnp.float32)
    part = lax.dot_general(
        P, h_ref[...], (((1,), (0,)), ((), ())),
        preferred_element_type=jnp.float32,
        precision=lax.Precision.HIGHEST,
    )
    cnt = jnp.sum(P, axis=1, keepdims=True)

    @pl.when(i == 0)
    def _():
        ps_scr[...] = part
        ct_scr[...] = jnp.broadcast_to(cnt, (NG, DH))

    @pl.when(i > 0)
    def _():
        ps_scr[...] += part
        ct_scr[...] += jnp.broadcast_to(cnt, (NG, DH))

    @pl.when(i == NBM - 1)
    def _():
        pooled = ps_scr[...] / jnp.maximum(ct_scr[...], 1.0)
        o1 = lax.dot_general(
            pooled, f1w_ref[...], (((1,), (0,)), ((), ())),
            preferred_element_type=jnp.float32,
            precision=lax.Precision.HIGHEST,
        )
        o1 = jnp.maximum(o1 + f1b_ref[...], 0.0)
        o2 = lax.dot_general(
            o1, f2w_ref[...], (((1,), (0,)), ((), ())),
            preferred_element_type=jnp.float32,
            precision=lax.Precision.HIGHEST,
        )
        o_ref[...] = o2 + f2b_ref[...]


def _tail(batch3, h, f1w, f1b, f2w, f2b):
    return pl.pallas_call(
        _tail_body,
        grid=(NBM,),
        in_specs=[
            pl.BlockSpec((1, 1, BM), lambda i: (i, 0, 0)),
            pl.BlockSpec((BM, DH), lambda i: (i, 0)),
            pl.BlockSpec((DH, DH), lambda i: (0, 0)),
            pl.BlockSpec((1, DH), lambda i: (0, 0)),
            pl.BlockSpec((DH, DOUT), lambda i: (0, 0)),
            pl.BlockSpec((1, DOUT), lambda i: (0, 0)),
        ],
        out_specs=pl.BlockSpec((NG, DOUT), lambda i: (0, 0)),
        out_shape=jax.ShapeDtypeStruct((NG, DOUT), jnp.float32),
        scratch_shapes=[
            pltpu.VMEM((NG, DH), jnp.float32),
            pltpu.VMEM((NG, DH), jnp.float32),
        ],
    )(batch3, h, f1w, f1b, f2w, f2b)


def kernel(x, edge_index, batch, W_in, b_in,
           W_c0, b_c0, gamma0, beta0,
           W_c1, b_c1, gamma1, beta1,
           W_c2, b_c2, gamma2, beta2,
           fc1_W, fc1_b, fc2_W, fc2_b):
    row = edge_index[0]
    col = edge_index[1]
    zeros_nd = jnp.zeros((N, DH), jnp.float32)
    zeros_nw = jnp.zeros((N, DW), jnp.float32)
    ones_cb = jnp.ones((CB, DW), jnp.float32)
    ones_n1 = jnp.ones((N, 1), jnp.float32)
    zero_b = jnp.zeros((1, DH), jnp.float32)

    degp = _deg_call(col, zeros_nw, ones_cb)
    dinv = _dinv_call(degp)[:, 0:1]

    h = _mm(x, W_in, b_in.reshape(1, DH), ones_n1)

    layers = [(W_c0, b_c0, gamma0, beta0),
              (W_c1, b_c1, gamma1, beta1),
              (W_c2, b_c2, gamma2, beta2)]
    for (W, b, g, be) in layers:
        hws = _mm(h, W, zero_b, dinv)
        agg = _seg_call(hws, row, col, zeros_nd)
        h = _post(agg, hws, dinv, b.reshape(1, DH), g.reshape(1, DH),
                  be.reshape(1, DH), h)

    batch3 = batch.reshape(NBM, 1, BM)
    out = _tail(batch3, h, fc1_W, fc1_b.reshape(1, DH),
                fc2_W, fc2_b.reshape(1, DOUT))
    return out


# speedup vs baseline: 21.8390x; 1.0365x over previous
import jax
import jax.numpy as jnp
from jax import lax
from jax.experimental import pallas as pl
from jax.experimental.pallas import tpu as pltpu
from jax.experimental.pallas import tpu_sc as plsc

N = 10000
E = 320000
DH = 128
D_IN = 128
NG = 64
DOUT = 16

NC = 2
NS = 16
NW = NC * NS
EPT = E // NW
CB = 40
NCHUNK = EPT // CB
G = 5
NGRP = NCHUNK // G
NWT = 10
WRT = N // NWT
DW = 128

BM = 1000
NBM = N // BM

_mesh = plsc.VectorSubcoreMesh(core_axis_name="c", subcore_axis_name="s",
                               num_cores=NC, num_subcores=NS)


def _deg_body(colr, zeros16, ones16, out, cidx, ones_v, deg, sem):
    c = lax.axis_index("c")
    s = lax.axis_index("s")
    wid = c * NS + s
    pltpu.sync_copy(colr.at[pl.ds(wid * EPT, EPT)], cidx)

    @pl.when(s < NWT)
    def _():
        pltpu.sync_copy(zeros16.at[pl.ds(s * WRT, WRT)],
                        deg.at[pl.ds(s * WRT, WRT)])

    pltpu.sync_copy(ones16, ones_v)
    plsc.subcore_barrier()

    def grp(g, carry):
        for j in range(G):
            pltpu.async_copy(
                ones_v, deg.at[cidx.at[pl.ds((g * G + j) * CB, CB)]],
                sem, add=True)
        for j in range(G):
            pltpu.make_async_copy(
                ones_v, deg.at[cidx.at[pl.ds(0, CB)]], sem).wait()
        return carry

    lax.fori_loop(0, NGRP, grp, 0)
    plsc.subcore_barrier()

    @pl.when(s < NWT)
    def _():
        pltpu.sync_copy(deg.at[pl.ds(s * WRT, WRT)],
                        out.at[c, pl.ds(s * WRT, WRT)])


_deg_call = pl.kernel(
    _deg_body,
    out_type=jax.ShapeDtypeStruct((NC, N, DW), jnp.float32),
    mesh=_mesh,
    scratch_types=[
        pltpu.VMEM((EPT,), jnp.int32),
        pltpu.VMEM((CB, DW), jnp.float32),
        pltpu.VMEM_SHARED((N, DW), jnp.float32),
        pltpu.SemaphoreType.DMA,
    ],
)


def _seg_body(hws, rowr, colr, zeros, out, ridx, cidx,
              r0, r1, r2, r3, r4, agg,
              g0, g1, g2, g3, g4, s0, s1, s2, s3, s4):
    rows = [r0, r1, r2, r3, r4]
    gsem = [g0, g1, g2, g3, g4]
    ssem = [s0, s1, s2, s3, s4]
    c = lax.axis_index("c")
    s = lax.axis_index("s")
    wid = c * NS + s
    pltpu.sync_copy(rowr.at[pl.ds(wid * EPT, EPT)], ridx)
    pltpu.sync_copy(colr.at[pl.ds(wid * EPT, EPT)], cidx)

    @pl.when(s < NWT)
    def _():
        pltpu.sync_copy(zeros.at[pl.ds(s * WRT, WRT)],
                        agg.at[pl.ds(s * WRT, WRT)])

    plsc.subcore_barrier()

    for j in range(G):
        pltpu.async_copy(hws.at[ridx.at[pl.ds(j * CB, CB)]], rows[j], gsem[j])

    def grp(g, carry):
        for j in range(G):
            k = g * G + j
            pltpu.make_async_copy(hws.at[ridx.at[pl.ds(k * CB, CB)]],
                                  rows[j], gsem[j]).wait()
            pltpu.async_copy(rows[j], agg.at[cidx.at[pl.ds(k * CB, CB)]],
                             ssem[j], add=True)

        @pl.when(g < NGRP - 1)
        def _():
            for j in range(G):
                k = g * G + j
                pltpu.make_async_copy(rows[j],
                                      agg.at[cidx.at[pl.ds(k * CB, CB)]],
                                      ssem[j]).wait()
                pltpu.async_copy(hws.at[ridx.at[pl.ds((k + G) * CB, CB)]],
                                 rows[j], gsem[j])

        return carry

    lax.fori_loop(0, NGRP, grp, 0)
    for j in range(G):
        pltpu.make_async_copy(rows[j], agg.at[cidx.at[pl.ds(0, CB)]],
                              ssem[j]).wait()
    plsc.subcore_barrier()

    @pl.when(s < NWT)
    def _():
        pltpu.sync_copy(agg.at[pl.ds(s * WRT, WRT)],
                        out.at[c, pl.ds(s * WRT, WRT)])


_seg_call = pl.kernel(
    _seg_body,
    out_type=jax.ShapeDtypeStruct((NC, N, DH), jnp.float32),
    mesh=_mesh,
    scratch_types=[
        pltpu.VMEM((EPT,), jnp.int32),
        pltpu.VMEM((EPT,), jnp.int32),
    ] + [pltpu.VMEM((CB, DH), jnp.float32) for _ in range(G)] + [
        pltpu.VMEM_SHARED((N, DH), jnp.float32),
    ] + [pltpu.SemaphoreType.DMA for _ in range(2 * G)],
)


def _dot(a, b):
    return lax.dot_general(
        a, b, (((1,), (0,)), ((), ())),
        preferred_element_type=jnp.float32,
        precision=lax.Precision.HIGHEST,
    )


def _projmm_body(x_ref, wi_ref, bi_ref, w0_ref, d_ref, oh_ref, ohws_ref):
    h = _dot(x_ref[...], wi_ref[...]) + bi_ref[...]
    oh_ref[...] = h
    ohws_ref[...] = _dot(h, w0_ref[...]) * d_ref[...]


def _projmm(x, wi, bi, w0, dinv):
    return pl.pallas_call(
        _projmm_body,
        grid=(NBM,),
        in_specs=[
            pl.BlockSpec((BM, D_IN), lambda i: (i, 0)),
            pl.BlockSpec((D_IN, DH), lambda i: (0, 0)),
            pl.BlockSpec((1, DH), lambda i: (0, 0)),
            pl.BlockSpec((DH, DH), lambda i: (0, 0)),
            pl.BlockSpec((BM, 1), lambda i: (i, 0)),
        ],
        out_specs=[
            pl.BlockSpec((BM, DH), lambda i: (i, 0)),
            pl.BlockSpec((BM, DH), lambda i: (i, 0)),
        ],
        out_shape=[
            jax.ShapeDtypeStruct((N, DH), jnp.float32),
            jax.ShapeDtypeStruct((N, DH), jnp.float32),
        ],
    )(x, wi, bi, w0, dinv)


def _dinv_body(degp_ref, o_ref):
    deg = degp_ref[0] + degp_ref[1] + 2.0
    o_ref[...] = lax.rsqrt(deg)


def _dinv_call(degp):
    return pl.pallas_call(
        _dinv_body,
        out_shape=jax.ShapeDtypeStruct((N, DW), jnp.float32),
    )(degp)


def _bn_phase0(agg_ref, hws_ref, d_ref, b_ref, i, t_scr, s_scr, ss_scr):
    t = d_ref[...] * (agg_ref[0] + agg_ref[1] + 2.0 * hws_ref[...]) + b_ref[...]
    t_scr[pl.ds(i * BM, BM), :] = t
    ps = jnp.sum(t, axis=0, keepdims=True)
    pss = jnp.sum(t * t, axis=0, keepdims=True)

    @pl.when(i == 0)
    def _():
        s_scr[0:1, :] = ps
        ss_scr[0:1, :] = pss

    @pl.when(i > 0)
    def _():
        s_scr[0:1, :] += ps
        ss_scr[0:1, :] += pss


def _bn_phase1(g_ref, be_ref, h_ref, i, t_scr, s_scr, ss_scr):
    m = s_scr[0:1, :] / N
    v = ss_scr[0:1, :] / N - m * m
    t = t_scr[pl.ds(i * BM, BM), :]
    bn = (t - m) * lax.rsqrt(v + 1e-5) * g_ref[...] + be_ref[...]
    return jnp.maximum(bn, 0.0) + h_ref[...]


def _postmm_body(agg_ref, hws_ref, d_ref, b_ref, g_ref, be_ref, h_ref, w_ref,
                 oh_ref, ohws_ref, t_scr, s_scr, ss_scr):
    p = pl.program_id(0)
    i = pl.program_id(1)

    @pl.when(p == 0)
    def _():
        _bn_phase0(agg_ref, hws_ref, d_ref, b_ref, i, t_scr, s_scr, ss_scr)

    @pl.when(p == 1)
    def _():
        hn = _bn_phase1(g_ref, be_ref, h_ref, i, t_scr, s_scr, ss_scr)
        oh_ref[...] = hn
        ohws_ref[...] = _dot(hn, w_ref[...]) * d_ref[...]


def _postmm(agg, hws, dinv, b, g, be, h, w_next):
    return pl.pallas_call(
        _postmm_body,
        grid=(2, NBM),
        in_specs=[
            pl.BlockSpec((NC, BM, DH), lambda p, i: (0, (1 - p) * i, 0)),
            pl.BlockSpec((BM, DH), lambda p, i: ((1 - p) * i, 0)),
            pl.BlockSpec((BM, 1), lambda p, i: (i, 0)),
            pl.BlockSpec((1, DH), lambda p, i: (0, 0)),
            pl.BlockSpec((1, DH), lambda p, i: (0, 0)),
            pl.BlockSpec((1, DH), lambda p, i: (0, 0)),
            pl.BlockSpec((BM, DH), lambda p, i: (p * i, 0)),
            pl.BlockSpec((DH, DH), lambda p, i: (0, 0)),
        ],
        out_specs=[
            pl.BlockSpec((BM, DH), lambda p, i: (p * i, 0)),
            pl.BlockSpec((BM, DH), lambda p, i: (p * i, 0)),
        ],
        out_shape=[
            jax.ShapeDtypeStruct((N, DH), jnp.float32),
            jax.ShapeDtypeStruct((N, DH), jnp.float32),
        ],
        scratch_shapes=[
            pltpu.VMEM((N, DH), jnp.float32),
            pltpu.VMEM((8, DH), jnp.float32),
            pltpu.VMEM((8, DH), jnp.float32),
        ],
    )(agg, hws, dinv, b, g, be, h, w_next)


def _posttail_body(agg_ref, hws_ref, d_ref, b_ref, g_ref, be_ref, h_ref,
                   b3_ref, f1w_ref, f1b_ref, f2w_ref, f2b_ref, o_ref,
                   t_scr, s_scr, ss_scr, ps_scr, ct_scr):
    p = pl.program_id(0)
    i = pl.program_id(1)

    @pl.when(p == 0)
    def _():
        _bn_phase0(agg_ref, hws_ref, d_ref, b_ref, i, t_scr, s_scr, ss_scr)

    @pl.when(p == 1)
    def _():
        hn = _bn_phase1(g_ref, be_ref, h_ref, i, t_scr, s_scr, ss_scr)
        bb = b3_ref[0]
        gid = lax.broadcasted_iota(jnp.int32, (NG, BM), 0)
        P = (jnp.broadcast_to(bb, (NG, BM)) == gid).astype(jnp.float32)
        part = _dot(P, hn)
        cnt = jnp.sum(P, axis=1, keepdims=True)

        @pl.when(i == 0)
        def _():
            ps_scr[...] = part
            ct_scr[...] = jnp.broadcast_to(cnt, (NG, DH))

        @pl.when(i > 0)
        def _():
            ps_scr[...] += part
            ct_scr[...] += jnp.broadcast_to(cnt, (NG, DH))

        @pl.when(i == NBM - 1)
        def _():
            pooled = ps_scr[...] / jnp.maximum(ct_scr[...], 1.0)
            o1 = jnp.maximum(_dot(pooled, f1w_ref[...]) + f1b_ref[...], 0.0)
            o_ref[...] = _dot(o1, f2w_ref[...]) + f2b_ref[...]


def _posttail(agg, hws, dinv, b, g, be, h, batch3, f1w, f1b, f2w, f2b):
    return pl.pallas_call(
        _posttail_body,
        grid=(2, NBM),
        in_specs=[
            pl.BlockSpec((NC, BM, DH), lambda p, i: (0, (1 - p) * i, 0)),
            pl.BlockSpec((BM, DH), lambda p, i: ((1 - p) * i, 0)),
            pl.BlockSpec((BM, 1), lambda p, i: (i, 0)),
            pl.BlockSpec((1, DH), lambda p, i: (0, 0)),
            pl.BlockSpec((1, DH), lambda p, i: (0, 0)),
            pl.BlockSpec((1, DH), lambda p, i: (0, 0)),
            pl.BlockSpec((BM, DH), lambda p, i: (p * i, 0)),
            pl.BlockSpec((1, 1, BM), lambda p, i: (p * i, 0, 0)),
            pl.BlockSpec((DH, DH), lambda p, i: (0, 0)),
            pl.BlockSpec((1, DH), lambda p, i: (0, 0)),
            pl.BlockSpec((DH, DOUT), lambda p, i: (0, 0)),
            pl.BlockSpec((1, DOUT), lambda p, i: (0, 0)),
        ],
        out_specs=pl.BlockSpec((NG, DOUT), lambda p, i: (0, 0)),
        out_shape=jax.ShapeDtypeStruct((NG, DOUT), jnp.float32),
        scratch_shapes=[
            pltpu.VMEM((N, DH), jnp.float32),
            pltpu.VMEM((8, DH), jnp.float32),
            pltpu.VMEM((8, DH), jnp.float32),
            pltpu.VMEM((NG, DH), jnp.float32),
            pltpu.VMEM((NG, DH), jnp.float32),
        ],
    )(agg, hws, dinv, b, g, be, h, batch3, f1w, f1b, f2w, f2b)


def kernel(x, edge_index, batch, W_in, b_in,
           W_c0, b_c0, gamma0, beta0,
           W_c1, b_c1, gamma1, beta1,
           W_c2, b_c2, gamma2, beta2,
           fc1_W, fc1_b, fc2_W, fc2_b):
    row = edge_index[0]
    col = edge_index[1]
    zeros_nd = jnp.zeros((N, DH), jnp.float32)
    ones_cb = jnp.ones((CB, DW), jnp.float32)

    degp = _deg_call(col, zeros_nd, ones_cb)
    dinv = _dinv_call(degp)[:, 0:1]

    h, hws = _projmm(x, W_in, b_in.reshape(1, DH), W_c0, dinv)

    agg = _seg_call(hws, row, col, zeros_nd)
    h, hws = _postmm(agg, hws, dinv, b_c0.reshape(1, DH),
                     gamma0.reshape(1, DH), beta0.reshape(1, DH), h, W_c1)

    agg = _seg_call(hws, row, col, zeros_nd)
    h, hws = _postmm(agg, hws, dinv, b_c1.reshape(1, DH),
                     gamma1.reshape(1, DH), beta1.reshape(1, DH), h, W_c2)

    agg = _seg_call(hws, row, col, zeros_nd)
    batch3 = batch.reshape(NBM, 1, BM)
    out = _posttail(agg, hws, dinv, b_c2.reshape(1, DH),
                    gamma2.reshape(1, DH), beta2.reshape(1, DH), h, batch3,
                    fc1_W, fc1_b.reshape(1, DH), fc2_W,
                    fc2_b.reshape(1, DOUT))
    return out


# speedup vs baseline: 23.4736x; 1.0748x over previous
import jax
import jax.numpy as jnp
from jax import lax
from jax.experimental import pallas as pl
from jax.experimental.pallas import tpu as pltpu
from jax.experimental.pallas import tpu_sc as plsc

N = 10000
E = 320000
DH = 128
D_IN = 128
NG = 64
DOUT = 16

NC = 2
NS = 16
NW = NC * NS
EPT = E // NW
CB = 40
NCHUNK = EPT // CB
G = 5
NGRP = NCHUNK // G
NWT = 10
WRT = N // NWT

HHALF = 5120
NPAD = 10112
HCH = EPT // 16

BM = 1000
NBM = N // BM

_mesh = plsc.VectorSubcoreMesh(core_axis_name="c", subcore_axis_name="s",
                               num_cores=NC, num_subcores=NS)


def _deg_body(colr, out, cidx, hist, red):
    c = lax.axis_index("c")
    s = lax.axis_index("s")
    wid = c * NS + s
    pltpu.sync_copy(colr.at[pl.ds(wid * EPT, EPT)], cidx)

    lane = lax.broadcasted_iota(jnp.int32, (16,), 0)
    lanebase = lane * HHALF
    onev = jnp.full((16,), 1.0, jnp.float32)
    zerov = jnp.full((16,), 0.0, jnp.float32)

    for p in range(2):
        base = p * HHALF

        def zbody(i, carry):
            for u in range(8):
                hist[pl.ds((i * 8 + u) * 16, 16)] = zerov
            return carry

        lax.fori_loop(0, 16 * HHALF // (16 * 8), zbody, 0)

        def sbody(i, carry):
            for u in range(5):
                k = i * 5 + u
                colv = cidx[pl.ds(k * 16, 16)]
                cshift = colv - base
                m = (cshift >= 0) & (cshift < HHALF)
                plsc.addupdate_scatter(hist, [lanebase + cshift], onev,
                                       mask=m)
            return carry

        lax.fori_loop(0, HCH // 5, sbody, 0)

        nred = (HHALF if p == 0 else NPAD - HHALF) // 16

        def rbody(i, carry):
            acc = hist[pl.ds(i * 16, 16)]
            for j in range(1, 16):
                acc = acc + hist[pl.ds(j * HHALF + i * 16, 16)]
            red[pl.ds(base + i * 16, 16)] = acc
            return carry

        lax.fori_loop(0, nred, rbody, 0)

    pltpu.sync_copy(red, out.at[wid])


_deg_call = pl.kernel(
    _deg_body,
    out_type=jax.ShapeDtypeStruct((NW, NPAD), jnp.float32),
    mesh=_mesh,
    scratch_types=[
        pltpu.VMEM((EPT,), jnp.int32),
        pltpu.VMEM((16 * HHALF,), jnp.float32),
        pltpu.VMEM((NPAD,), jnp.float32),
    ],
    compiler_params=pltpu.CompilerParams(needs_layout_passes=False),
)


def _seg_body(hws, rowr, colr, zeros, out, ridx, cidx,
              r0, r1, r2, r3, r4, agg,
              g0, g1, g2, g3, g4, s0, s1, s2, s3, s4):
    rows = [r0, r1, r2, r3, r4]
    gsem = [g0, g1, g2, g3, g4]
    ssem = [s0, s1, s2, s3, s4]
    c = lax.axis_index("c")
    s = lax.axis_index("s")
    wid = c * NS + s
    pltpu.sync_copy(rowr.at[pl.ds(wid * EPT, EPT)], ridx)
    pltpu.sync_copy(colr.at[pl.ds(wid * EPT, EPT)], cidx)

    @pl.when(s < NWT)
    def _():
        pltpu.sync_copy(zeros.at[pl.ds(s * WRT, WRT)],
                        agg.at[pl.ds(s * WRT, WRT)])

    plsc.subcore_barrier()

    for j in range(G):
        pltpu.async_copy(hws.at[ridx.at[pl.ds(j * CB, CB)]], rows[j], gsem[j])

    def grp(g, carry):
        for j in range(G):
            k = g * G + j
            pltpu.make_async_copy(hws.at[ridx.at[pl.ds(k * CB, CB)]],
                                  rows[j], gsem[j]).wait()
            pltpu.async_copy(rows[j], agg.at[cidx.at[pl.ds(k * CB, CB)]],
                             ssem[j], add=True)

        @pl.when(g < NGRP - 1)
        def _():
            for j in range(G):
                k = g * G + j
                pltpu.make_async_copy(rows[j],
                                      agg.at[cidx.at[pl.ds(k * CB, CB)]],
                                      ssem[j]).wait()
                pltpu.async_copy(hws.at[ridx.at[pl.ds((k + G) * CB, CB)]],
                                 rows[j], gsem[j])

        return carry

    lax.fori_loop(0, NGRP, grp, 0)
    for j in range(G):
        pltpu.make_async_copy(rows[j], agg.at[cidx.at[pl.ds(0, CB)]],
                              ssem[j]).wait()
    plsc.subcore_barrier()

    @pl.when(s < NWT)
    def _():
        pltpu.sync_copy(agg.at[pl.ds(s * WRT, WRT)],
                        out.at[c, pl.ds(s * WRT, WRT)])


_seg_call = pl.kernel(
    _seg_body,
    out_type=jax.ShapeDtypeStruct((NC, N, DH), jnp.float32),
    mesh=_mesh,
    scratch_types=[
        pltpu.VMEM((EPT,), jnp.int32),
        pltpu.VMEM((EPT,), jnp.int32),
    ] + [pltpu.VMEM((CB, DH), jnp.float32) for _ in range(G)] + [
        pltpu.VMEM_SHARED((N, DH), jnp.float32),
    ] + [pltpu.SemaphoreType.DMA for _ in range(2 * G)],
)


def _dot(a, b):
    return lax.dot_general(
        a, b, (((1,), (0,)), ((), ())),
        preferred_element_type=jnp.float32,
        precision=lax.Precision.HIGHEST,
    )


def _projmm_body(x_ref, wi_ref, bi_ref, w0_ref, d_ref, oh_ref, ohws_ref):
    h = _dot(x_ref[...], wi_ref[...]) + bi_ref[...]
    oh_ref[...] = h
    ohws_ref[...] = _dot(h, w0_ref[...]) * d_ref[...]


def _projmm(x, wi, bi, w0, dinv):
    return pl.pallas_call(
        _projmm_body,
        grid=(NBM,),
        in_specs=[
            pl.BlockSpec((BM, D_IN), lambda i: (i, 0)),
            pl.BlockSpec((D_IN, DH), lambda i: (0, 0)),
            pl.BlockSpec((1, DH), lambda i: (0, 0)),
            pl.BlockSpec((DH, DH), lambda i: (0, 0)),
            pl.BlockSpec((BM, 1), lambda i: (i, 0)),
        ],
        out_specs=[
            pl.BlockSpec((BM, DH), lambda i: (i, 0)),
            pl.BlockSpec((BM, DH), lambda i: (i, 0)),
        ],
        out_shape=[
            jax.ShapeDtypeStruct((N, DH), jnp.float32),
            jax.ShapeDtypeStruct((N, DH), jnp.float32),
        ],
    )(x, wi, bi, w0, dinv)


def _dinv_body(degp_ref, o_ref):
    deg = jnp.sum(degp_ref[...], axis=0, keepdims=True) + 2.0
    o_ref[...] = lax.rsqrt(deg)


def _dinv_call(degp):
    return pl.pallas_call(
        _dinv_body,
        out_shape=jax.ShapeDtypeStruct((1, NPAD), jnp.float32),
    )(degp)


def _bn_phase0(agg_ref, hws_ref, d_ref, b_ref, i, t_scr, s_scr, ss_scr):
    t = d_ref[...] * (agg_ref[0] + agg_ref[1] + 2.0 * hws_ref[...]) + b_ref[...]
    t_scr[pl.ds(i * BM, BM), :] = t
    ps = jnp.sum(t, axis=0, keepdims=True)
    pss = jnp.sum(t * t, axis=0, keepdims=True)

    @pl.when(i == 0)
    def _():
        s_scr[0:1, :] = ps
        ss_scr[0:1, :] = pss

    @pl.when(i > 0)
    def _():
        s_scr[0:1, :] += ps
        ss_scr[0:1, :] += pss


def _bn_phase1(g_ref, be_ref, h_ref, i, t_scr, s_scr, ss_scr):
    m = s_scr[0:1, :] / N
    v = ss_scr[0:1, :] / N - m * m
    t = t_scr[pl.ds(i * BM, BM), :]
    bn = (t - m) * lax.rsqrt(v + 1e-5) * g_ref[...] + be_ref[...]
    return jnp.maximum(bn, 0.0) + h_ref[...]


def _postmm_body(agg_ref, hws_ref, d_ref, b_ref, g_ref, be_ref, h_ref, w_ref,
                 oh_ref, ohws_ref, t_scr, s_scr, ss_scr):
    p = pl.program_id(0)
    i = pl.program_id(1)

    @pl.when(p == 0)
    def _():
        _bn_phase0(agg_ref, hws_ref, d_ref, b_ref, i, t_scr, s_scr, ss_scr)

    @pl.when(p == 1)
    def _():
        hn = _bn_phase1(g_ref, be_ref, h_ref, i, t_scr, s_scr, ss_scr)
        oh_ref[...] = hn
        ohws_ref[...] = _dot(hn, w_ref[...]) * d_ref[...]


def _postmm(agg, hws, dinv, b, g, be, h, w_next):
    return pl.pallas_call(
        _postmm_body,
        grid=(2, NBM),
        in_specs=[
            pl.BlockSpec((NC, BM, DH), lambda p, i: (0, (1 - p) * i, 0)),
            pl.BlockSpec((BM, DH), lambda p, i: ((1 - p) * i, 0)),
            pl.BlockSpec((BM, 1), lambda p, i: (i, 0)),
            pl.BlockSpec((1, DH), lambda p, i: (0, 0)),
            pl.BlockSpec((1, DH), lambda p, i: (0, 0)),
            pl.BlockSpec((1, DH), lambda p, i: (0, 0)),
            pl.BlockSpec((BM, DH), lambda p, i: (p * i, 0)),
            pl.BlockSpec((DH, DH), lambda p, i: (0, 0)),
        ],
        out_specs=[
            pl.BlockSpec((BM, DH), lambda p, i: (p * i, 0)),
            pl.BlockSpec((BM, DH), lambda p, i: (p * i, 0)),
        ],
        out_shape=[
            jax.ShapeDtypeStruct((N, DH), jnp.float32),
            jax.ShapeDtypeStruct((N, DH), jnp.float32),
        ],
        scratch_shapes=[
            pltpu.VMEM((N, DH), jnp.float32),
            pltpu.VMEM((8, DH), jnp.float32),
            pltpu.VMEM((8, DH), jnp.float32),
        ],
    )(agg, hws, dinv, b, g, be, h, w_next)


def _posttail_body(agg_ref, hws_ref, d_ref, b_ref, g_ref, be_ref, h_ref,
                   b3_ref, f1w_ref, f1b_ref, f2w_ref, f2b_ref, o_ref,
                   t_scr, s_scr, ss_scr, ps_scr, ct_scr):
    p = pl.program_id(0)
    i = pl.program_id(1)

    @pl.when(p == 0)
    def _():
        _bn_phase0(agg_ref, hws_ref, d_ref, b_ref, i, t_scr, s_scr, ss_scr)

    @pl.when(p == 1)
    def _():
        hn = _bn_phase1(g_ref, be_ref, h_ref, i, t_scr, s_scr, ss_scr)
        bb = b3_ref[0]
        gid = lax.broadcasted_iota(jnp.int32, (NG, BM), 0)
        P = (jnp.broadcast_to(bb, (NG, BM)) == gid).astype(jnp.float32)
        part = _dot(P, hn)
        cnt = jnp.sum(P, axis=1, keepdims=True)

        @pl.when(i == 0)
        def _():
            ps_scr[...] = part
            ct_scr[...] = jnp.broadcast_to(cnt, (NG, DH))

        @pl.when(i > 0)
        def _():
            ps_scr[...] += part
            ct_scr[...] += jnp.broadcast_to(cnt, (NG, DH))

        @pl.when(i == NBM - 1)
        def _():
            pooled = ps_scr[...] / jnp.maximum(ct_scr[...], 1.0)
            o1 = jnp.maximum(_dot(pooled, f1w_ref[...]) + f1b_ref[...], 0.0)
            o_ref[...] = _dot(o1, f2w_ref[...]) + f2b_ref[...]


def _posttail(agg, hws, dinv, b, g, be, h, batch3, f1w, f1b, f2w, f2b):
    return pl.pallas_call(
        _posttail_body,
        grid=(2, NBM),
        in_specs=[
            pl.BlockSpec((NC, BM, DH), lambda p, i: (0, (1 - p) * i, 0)),
            pl.BlockSpec((BM, DH), lambda p, i: ((1 - p) * i, 0)),
            pl.BlockSpec((BM, 1), lambda p, i: (i, 0)),
            pl.BlockSpec((1, DH), lambda p, i: (0, 0)),
            pl.BlockSpec((1, DH), lambda p, i: (0, 0)),
            pl.BlockSpec((1, DH), lambda p, i: (0, 0)),
            pl.BlockSpec((BM, DH), lambda p, i: (p * i, 0)),
            pl.BlockSpec((1, 1, BM), lambda p, i: (p * i, 0, 0)),
            pl.BlockSpec((DH, DH), lambda p, i: (0, 0)),
            pl.BlockSpec((1, DH), lambda p, i: (0, 0)),
            pl.BlockSpec((DH, DOUT), lambda p, i: (0, 0)),
            pl.BlockSpec((1, DOUT), lambda p, i: (0, 0)),
        ],
        out_specs=pl.BlockSpec((NG, DOUT), lambda p, i: (0, 0)),
        out_shape=jax.ShapeDtypeStruct((NG, DOUT), jnp.float32),
        scratch_shapes=[
            pltpu.VMEM((N, DH), jnp.float32),
            pltpu.VMEM((8, DH), jnp.float32),
            pltpu.VMEM((8, DH), jnp.float32),
            pltpu.VMEM((NG, DH), jnp.float32),
            pltpu.VMEM((NG, DH), jnp.float32),
        ],
    )(agg, hws, dinv, b, g, be, h, batch3, f1w, f1b, f2w, f2b)


def kernel(x, edge_index, batch, W_in, b_in,
           W_c0, b_c0, gamma0, beta0,
           W_c1, b_c1, gamma1, beta1,
           W_c2, b_c2, gamma2, beta2,
           fc1_W, fc1_b, fc2_W, fc2_b):
    row = edge_index[0]
    col = edge_index[1]
    zeros_nd = jnp.zeros((N, DH), jnp.float32)

    degp = _deg_call(col)
    dinv = _dinv_call(degp).reshape(NPAD, 1)[:N]

    h, hws = _projmm(x, W_in, b_in.reshape(1, DH), W_c0, dinv)

    agg = _seg_call(hws, row, col, zeros_nd)
    h, hws = _postmm(agg, hws, dinv, b_c0.reshape(1, DH),
                     gamma0.reshape(1, DH), beta0.reshape(1, DH), h, W_c1)

    agg = _seg_call(hws, row, col, zeros_nd)
    h, hws = _postmm(agg, hws, dinv, b_c1.reshape(1, DH),
                     gamma1.reshape(1, DH), beta1.reshape(1, DH), h, W_c2)

    agg = _seg_call(hws, row, col, zeros_nd)
    batch3 = batch.reshape(NBM, 1, BM)
    out = _posttail(agg, hws, dinv, b_c2.reshape(1, DH),
                    gamma2.reshape(1, DH), beta2.reshape(1, DH), h, batch3,
                    fc1_W, fc1_b.reshape(1, DH), fc2_W,
                    fc2_b.reshape(1, DOUT))
    return out
